# baseline (speedup 1.0000x reference)
import functools

import jax
import jax.numpy as jnp
from jax import lax
from jax.experimental import pallas as pl
from jax.experimental.pallas import tpu as pltpu
from jax.experimental.pallas import tpu_sc as plsc

N = 10000
E = 320000
D = 128
SLOPE = 0.01

NC, NS = 2, 16
NW = NC * NS
G = 125
EPW = E // NW
NCHUNK = EPW // G
NPAD = 10240
DEG_SLICE = NPAD // NS
ROWS_PER_TILE = NPAD // NS
RCHUNK = 64

_mesh = plsc.VectorSubcoreMesh(core_axis_name="c", subcore_axis_name="s")


@functools.partial(
    pl.kernel,
    out_type=tuple(jax.ShapeDtypeStruct((NPAD,), jnp.float32) for _ in range(4)),
    mesh=_mesh,
    scratch_types=[
        pltpu.VMEM((NCHUNK, G), jnp.int32),
        pltpu.VMEM((NCHUNK, G), jnp.int32),
        pltpu.VMEM((128,), jnp.float32),
        pltpu.VMEM((DEG_SLICE,), jnp.float32),
        pltpu.VMEM_SHARED((NPAD,), jnp.float32),
        pltpu.VMEM_SHARED((NPAD,), jnp.float32),
    ],
)
def _degrees_kernel(src_hbm, dst_hbm, d00_hbm, d01_hbm, d10_hbm, d11_hbm,
                    src_v, dst_v, ones_v, buf_v, hout_sh, hin_sh):
    c = lax.axis_index("c")
    s = lax.axis_index("s")
    w = s * NC + c

    def fill_zero(i, carry):
        buf_v[pl.ds(i * 16, 16)] = jnp.zeros((16,), jnp.float32)
        return carry

    lax.fori_loop(0, DEG_SLICE // 16, fill_zero, 0)

    def fill_one(i, carry):
        ones_v[pl.ds(i * 16, 16)] = jnp.ones((16,), jnp.float32)
        return carry

    lax.fori_loop(0, 128 // 16, fill_one, 0)

    sl = pl.ds(s * DEG_SLICE, DEG_SLICE)
    pltpu.sync_copy(buf_v, hout_sh.at[sl])
    pltpu.sync_copy(buf_v, hin_sh.at[sl])
    pltpu.sync_copy(src_hbm.at[pl.ds(w * NCHUNK, NCHUNK), :], src_v)
    pltpu.sync_copy(dst_hbm.at[pl.ds(w * NCHUNK, NCHUNK), :], dst_v)
    plsc.subcore_barrier()

    ones_sl = ones_v.at[pl.ds(0, G)]

    def body(j, carry):
        pltpu.sync_copy(ones_sl, hout_sh.at[src_v.at[j]], add=True)
        pltpu.sync_copy(ones_sl, hin_sh.at[dst_v.at[j]], add=True)
        return carry

    lax.fori_loop(0, NCHUNK, body, 0)
    plsc.subcore_barrier()

    @pl.when(c == 0)
    def _():
        pltpu.sync_copy(hout_sh.at[sl], buf_v)
        pltpu.sync_copy(buf_v, d00_hbm.at[sl])
        pltpu.sync_copy(hin_sh.at[sl], buf_v)
        pltpu.sync_copy(buf_v, d01_hbm.at[sl])

    @pl.when(c == 1)
    def _():
        pltpu.sync_copy(hout_sh.at[sl], buf_v)
        pltpu.sync_copy(buf_v, d10_hbm.at[sl])
        pltpu.sync_copy(hin_sh.at[sl], buf_v)
        pltpu.sync_copy(buf_v, d11_hbm.at[sl])


GA = 40
PAIR = 2 * GA
NPAIR = EPW // PAIR


@functools.partial(
    pl.kernel,
    out_type=jax.ShapeDtypeStruct((NC, NPAD, D), jnp.float32),
    mesh=_mesh,
    scratch_types=[
        pltpu.VMEM((EPW,), jnp.int32),
        pltpu.VMEM((NPAIR, PAIR), jnp.int32),
        tuple(pltpu.VMEM((PAIR, D), jnp.float32) for _ in range(2)),
        pltpu.VMEM_SHARED((NPAD, D), jnp.float32),
        tuple(tuple(pltpu.SemaphoreType.DMA for _ in range(2)) for _ in range(2)),
        tuple(pltpu.SemaphoreType.DMA for _ in range(2)),
    ],
)
def _aggregate_kernel(feat_hbm, src_hbm, dst_hbm, agg_hbm,
                      src_v, dst_v, rows, agg_sh, gsem, ssem):
    c = lax.axis_index("c")
    s = lax.axis_index("s")
    w = s * NC + c

    def fill_row(i, carry):
        def fill_col(k, carry2):
            rows[0][i, pl.ds(k * 16, 16)] = jnp.zeros((16,), jnp.float32)
            return carry2

        lax.fori_loop(0, D // 16, fill_col, 0)
        return carry

    lax.fori_loop(0, PAIR, fill_row, 0)

    base_row = s * ROWS_PER_TILE
    for r in range(ROWS_PER_TILE // PAIR):
        pltpu.sync_copy(rows[0], agg_sh.at[pl.ds(base_row + r * PAIR, PAIR), :])
    rem = ROWS_PER_TILE % PAIR
    if rem:
        pltpu.sync_copy(rows[0].at[pl.ds(0, rem), :],
                        agg_sh.at[pl.ds(base_row + (ROWS_PER_TILE // PAIR) * PAIR,
                                        rem), :])

    pltpu.sync_copy(src_hbm.at[pl.ds(w * EPW, EPW)], src_v)
    pltpu.sync_copy(dst_hbm.at[w], dst_v)
    plsc.subcore_barrier()

    def gather_pair(p, buf):
        for h in range(2):
            pltpu.async_copy(
                feat_hbm.at[src_v.at[pl.ds(p * PAIR + h * GA, GA)]],
                rows[buf].at[pl.ds(h * GA, GA), :], gsem[buf][h])

    def wait_pair(buf):
        for h in range(2):
            pltpu.make_async_copy(
                feat_hbm.at[src_v.at[pl.ds(h * GA, GA)]],
                rows[buf].at[pl.ds(h * GA, GA), :], gsem[buf][h]).wait()

    gather_pair(0, 0)

    def body(i, carry):
        for b in range(2):
            p = i * 2 + b
            wait_pair(b)

            @pl.when(p >= 1)
            def _():
                pltpu.make_async_copy(rows[1 - b], agg_sh.at[dst_v.at[0]],
                                      ssem[1 - b]).wait()

            @pl.when(p + 1 < NPAIR)
            def _():
                gather_pair(p + 1, 1 - b)

            pltpu.async_copy(rows[b], agg_sh.at[dst_v.at[p]], add=True,
                             sem=ssem[b])
        return carry

    lax.fori_loop(0, NPAIR // 2, body, 0)
    if NPAIR % 2:
        wait_pair(0)
        pltpu.make_async_copy(rows[1], agg_sh.at[dst_v.at[0]], ssem[1]).wait()
        pltpu.async_copy(rows[0], agg_sh.at[dst_v.at[NPAIR - 1]], add=True,
                         sem=ssem[0])
        pltpu.make_async_copy(rows[0], agg_sh.at[dst_v.at[0]], ssem[0]).wait()
    else:
        pltpu.make_async_copy(rows[1], agg_sh.at[dst_v.at[0]], ssem[1]).wait()
    plsc.subcore_barrier()

    sl = pl.ds(base_row, ROWS_PER_TILE)
    pltpu.sync_copy(agg_sh.at[sl, :], agg_hbm.at[c, sl, :])


RB = 1024
NBLK = NPAD // RB


def _feat_body(d00_ref, d10_ref, v_ref, feat_ref):
    d_out = d00_ref[...] + d10_ref[...]
    rs = lax.rsqrt(jnp.maximum(d_out, 1.0))
    feat_ref[...] = v_ref[...] * rs[:, None]


_feat_call = pl.pallas_call(
    _feat_body,
    grid=(NBLK,),
    in_specs=[
        pl.BlockSpec((RB,), lambda i: (i,)),
        pl.BlockSpec((RB,), lambda i: (i,)),
        pl.BlockSpec((RB, D), lambda i: (i, 0)),
    ],
    out_specs=pl.BlockSpec((RB, D), lambda i: (i, 0)),
    out_shape=jax.ShapeDtypeStruct((N, D), jnp.float32),
)


def _out_body(aggp_ref, d01_ref, d11_ref, v_ref, w_ref, b_ref, out_ref):
    agg = aggp_ref[0] + aggp_ref[1]
    d_in = d01_ref[...] + d11_ref[...]
    rs = lax.rsqrt(jnp.maximum(d_in, 1.0))
    rst = agg * rs[:, None]
    rst = jnp.dot(rst, w_ref[...], preferred_element_type=jnp.float32)
    rst = rst + b_ref[...]
    out_ref[...] = jnp.where(rst > 0, rst, SLOPE * rst) + v_ref[...]


_out_call = pl.pallas_call(
    _out_body,
    grid=(NBLK,),
    in_specs=[
        pl.BlockSpec((NC, RB, D), lambda i: (0, i, 0)),
        pl.BlockSpec((RB,), lambda i: (i,)),
        pl.BlockSpec((RB,), lambda i: (i,)),
        pl.BlockSpec((RB, D), lambda i: (i, 0)),
        pl.BlockSpec((D, D), lambda i: (0, 0)),
        pl.BlockSpec((1, D), lambda i: (0, 0)),
    ],
    out_specs=pl.BlockSpec((RB, D), lambda i: (i, 0)),
    out_shape=jax.ShapeDtypeStruct((N, D), jnp.float32),
)


def kernel(V, edge_index, W, b):
    src = edge_index[0].reshape(E // G, G)
    dst = edge_index[1].reshape(E // G, G)
    src_a = edge_index[0]
    dst_a = edge_index[1].reshape(NW, NPAIR, PAIR)
    d00, d01, d10, d11 = _degrees_kernel(src, dst)
    feat = _feat_call(d00, d10, V)
    aggp = _aggregate_kernel(feat, src_a, dst_a)
    return _out_call(aggp, d01, d11, V, W, b.reshape(1, D))


# speedup vs baseline: 1.1445x; 1.1445x over previous
import functools

import jax
import jax.numpy as jnp
from jax import lax
from jax.experimental import pallas as pl
from jax.experimental.pallas import tpu as pltpu
from jax.experimental.pallas import tpu_sc as plsc

N = 10000
E = 320000
D = 128
SLOPE = 0.01

NC, NS = 2, 16
NW = NC * NS
G = 125
EPW = E // NW
NCHUNK = EPW // G
NPAD = 10240
DEG_SLICE = NPAD // NS
ROWS_PER_TILE = NPAD // NS

_mesh = plsc.VectorSubcoreMesh(core_axis_name="c", subcore_axis_name="s")


@functools.partial(
    pl.kernel,
    out_type=tuple(jax.ShapeDtypeStruct((NPAD,), jnp.float32) for _ in range(4)),
    mesh=_mesh,
    scratch_types=[
        pltpu.VMEM((NCHUNK, G), jnp.int32),
        pltpu.VMEM((NCHUNK, G), jnp.int32),
        pltpu.VMEM((128,), jnp.float32),
        pltpu.VMEM((DEG_SLICE,), jnp.float32),
        pltpu.VMEM_SHARED((NPAD,), jnp.float32),
        pltpu.VMEM_SHARED((NPAD,), jnp.float32),
        pltpu.SemaphoreType.DMA,
        pltpu.SemaphoreType.DMA,
    ],
)
def _degrees_kernel(e2_hbm, d00_hbm, d01_hbm, d10_hbm, d11_hbm,
                    src_v, dst_v, ones_v, buf_v, hout_sh, hin_sh, sa, sb):
    c = lax.axis_index("c")
    s = lax.axis_index("s")
    w = s * NC + c

    def fill_zero(i, carry):
        buf_v[pl.ds(i * 16, 16)] = jnp.zeros((16,), jnp.float32)
        return carry

    lax.fori_loop(0, DEG_SLICE // 16, fill_zero, 0)

    def fill_one(i, carry):
        ones_v[pl.ds(i * 16, 16)] = jnp.ones((16,), jnp.float32)
        return carry

    lax.fori_loop(0, 128 // 16, fill_one, 0)

    sl = pl.ds(s * DEG_SLICE, DEG_SLICE)
    pltpu.sync_copy(buf_v, hout_sh.at[sl])
    pltpu.sync_copy(buf_v, hin_sh.at[sl])
    rows = pl.ds(w * NCHUNK, NCHUNK)
    pltpu.sync_copy(e2_hbm.at[0, rows, :], src_v)
    pltpu.sync_copy(e2_hbm.at[1, rows, :], dst_v)
    plsc.subcore_barrier()

    ones_sl = ones_v.at[pl.ds(0, G)]

    def body(j, carry):
        @pl.when(j >= 1)
        def _():
            pltpu.make_async_copy(ones_sl, hout_sh.at[src_v.at[0]], sa).wait()
            pltpu.make_async_copy(ones_sl, hin_sh.at[dst_v.at[0]], sb).wait()

        pltpu.async_copy(ones_sl, hout_sh.at[src_v.at[j]], sa, add=True)
        pltpu.async_copy(ones_sl, hin_sh.at[dst_v.at[j]], sb, add=True)
        return carry

    lax.fori_loop(0, NCHUNK, body, 0)
    pltpu.make_async_copy(ones_sl, hout_sh.at[src_v.at[0]], sa).wait()
    pltpu.make_async_copy(ones_sl, hin_sh.at[dst_v.at[0]], sb).wait()
    plsc.subcore_barrier()

    @pl.when(c == 0)
    def _():
        pltpu.sync_copy(hout_sh.at[sl], buf_v)
        pltpu.sync_copy(buf_v, d00_hbm.at[sl])
        pltpu.sync_copy(hin_sh.at[sl], buf_v)
        pltpu.sync_copy(buf_v, d01_hbm.at[sl])

    @pl.when(c == 1)
    def _():
        pltpu.sync_copy(hout_sh.at[sl], buf_v)
        pltpu.sync_copy(buf_v, d10_hbm.at[sl])
        pltpu.sync_copy(hin_sh.at[sl], buf_v)
        pltpu.sync_copy(buf_v, d11_hbm.at[sl])


@functools.partial(
    pl.kernel,
    out_type=jax.ShapeDtypeStruct((NC, NPAD, D), jnp.float32),
    mesh=_mesh,
    scratch_types=[
        pltpu.VMEM((NCHUNK, G), jnp.int32),
        pltpu.VMEM((8, G), jnp.int32),
        tuple(pltpu.VMEM((G, D), jnp.float32) for _ in range(2)),
        pltpu.VMEM_SHARED((NPAD, D), jnp.float32),
        tuple(pltpu.SemaphoreType.DMA for _ in range(2)),
        tuple(pltpu.SemaphoreType.DMA for _ in range(2)),
    ],
)
def _aggregate_kernel(feat_hbm, e2_hbm, agg_hbm,
                      src_v, dst_v, rows, agg_sh, gsem, ssem):
    c = lax.axis_index("c")
    s = lax.axis_index("s")
    w = s * NC + c

    def fill_row(i, carry):
        def fill_col(k, carry2):
            rows[0][i, pl.ds(k * 16, 16)] = jnp.zeros((16,), jnp.float32)
            return carry2

        lax.fori_loop(0, D // 16, fill_col, 0)
        return carry

    lax.fori_loop(0, G, fill_row, 0)

    base_row = s * ROWS_PER_TILE
    for r in range(ROWS_PER_TILE // G):
        pltpu.sync_copy(rows[0], agg_sh.at[pl.ds(base_row + r * G, G), :])
    rem = ROWS_PER_TILE % G
    if rem:
        pltpu.sync_copy(rows[0].at[pl.ds(0, rem), :],
                        agg_sh.at[pl.ds(base_row + (ROWS_PER_TILE // G) * G,
                                        rem), :])

    pltpu.sync_copy(e2_hbm.at[0, pl.ds(w * NCHUNK, NCHUNK), :], src_v)
    pltpu.sync_copy(e2_hbm.at[1, pl.ds(w * NCHUNK, 8), :], dst_v)
    plsc.subcore_barrier()

    pltpu.async_copy(feat_hbm.at[src_v.at[0]], rows[0], gsem[0])

    def body(i, carry):
        for b in range(2):
            t = i * 2 + b
            pltpu.make_async_copy(feat_hbm.at[src_v.at[0]], rows[b],
                                  gsem[b]).wait()

            @pl.when(t >= 1)
            def _():
                pltpu.make_async_copy(rows[1 - b], agg_sh.at[dst_v.at[0]],
                                      ssem[1 - b]).wait()

            if b == 0:
                @pl.when(jnp.logical_and(t % 8 == 0, t > 0))
                def _():
                    pltpu.sync_copy(
                        e2_hbm.at[1, pl.ds(w * NCHUNK + (t // 8) * 8, 8), :],
                        dst_v)

            @pl.when(t + 1 < NCHUNK)
            def _():
                pltpu.async_copy(feat_hbm.at[src_v.at[t + 1]], rows[1 - b],
                                 gsem[1 - b])

            pltpu.async_copy(rows[b], agg_sh.at[dst_v.at[t % 8]], add=True,
                             sem=ssem[b])
        return carry

    lax.fori_loop(0, NCHUNK // 2, body, 0)
    pltpu.make_async_copy(rows[1], agg_sh.at[dst_v.at[0]], ssem[1]).wait()
    plsc.subcore_barrier()

    sl = pl.ds(base_row, ROWS_PER_TILE)
    pltpu.sync_copy(agg_sh.at[sl, :], agg_hbm.at[c, sl, :])


RB = 1024
NBLK = NPAD // RB


def _feat_body(d00_ref, d10_ref, v_ref, feat_ref):
    d_out = d00_ref[...] + d10_ref[...]
    rs = lax.rsqrt(jnp.maximum(d_out, 1.0))
    feat_ref[...] = v_ref[...] * rs[:, None]


_feat_call = pl.pallas_call(
    _feat_body,
    grid=(NBLK,),
    in_specs=[
        pl.BlockSpec((RB,), lambda i: (i,)),
        pl.BlockSpec((RB,), lambda i: (i,)),
        pl.BlockSpec((RB, D), lambda i: (i, 0)),
    ],
    out_specs=pl.BlockSpec((RB, D), lambda i: (i, 0)),
    out_shape=jax.ShapeDtypeStruct((N, D), jnp.float32),
)


def _out_body(aggp_ref, d01_ref, d11_ref, v_ref, w_ref, b_ref, out_ref):
    agg = aggp_ref[0] + aggp_ref[1]
    d_in = d01_ref[...] + d11_ref[...]
    rs = lax.rsqrt(jnp.maximum(d_in, 1.0))
    rst = agg * rs[:, None]
    rst = jnp.dot(rst, w_ref[...], preferred_element_type=jnp.float32)
    rst = rst + b_ref[...]
    out_ref[...] = jnp.where(rst > 0, rst, SLOPE * rst) + v_ref[...]


_out_call = pl.pallas_call(
    _out_body,
    grid=(NBLK,),
    in_specs=[
        pl.BlockSpec((NC, RB, D), lambda i: (0, i, 0)),
        pl.BlockSpec((RB,), lambda i: (i,)),
        pl.BlockSpec((RB,), lambda i: (i,)),
        pl.BlockSpec((RB, D), lambda i: (i, 0)),
        pl.BlockSpec((D, D), lambda i: (0, 0)),
        pl.BlockSpec((1, D), lambda i: (0, 0)),
    ],
    out_specs=pl.BlockSpec((RB, D), lambda i: (i, 0)),
    out_shape=jax.ShapeDtypeStruct((N, D), jnp.float32),
)


def kernel(V, edge_index, W, b):
    e2 = edge_index.reshape(2, E // G, G)
    d00, d01, d10, d11 = _degrees_kernel(e2)
    feat = _feat_call(d00, d10, V)
    aggp = _aggregate_kernel(feat, e2)
    return _out_call(aggp, d01, d11, V, W, b.reshape(1, D))


# speedup vs baseline: 1.1731x; 1.0250x over previous
import functools

import jax
import jax.numpy as jnp
from jax import lax
from jax.experimental import pallas as pl
from jax.experimental.pallas import tpu as pltpu
from jax.experimental.pallas import tpu_sc as plsc

N = 10000
E = 320000
D = 128
SLOPE = 0.01

NC, NS = 2, 16
NW = NC * NS
G = 125
EPW = E // NW
NCHUNK = EPW // G
NPAD = 10240
DEG_SLICE = NPAD // NS
ROWS_PER_TILE = NPAD // NS

_mesh = plsc.VectorSubcoreMesh(core_axis_name="c", subcore_axis_name="s")


@functools.partial(
    pl.kernel,
    out_type=tuple(jax.ShapeDtypeStruct((NPAD,), jnp.float32) for _ in range(2)),
    mesh=_mesh,
    scratch_types=[
        pltpu.VMEM((NCHUNK, G), jnp.int32),
        pltpu.VMEM((128,), jnp.float32),
        pltpu.VMEM((DEG_SLICE,), jnp.float32),
        pltpu.VMEM_SHARED((NPAD,), jnp.float32),
        pltpu.SemaphoreType.DMA,
    ],
)
def _degrees_kernel(e2_hbm, d00_hbm, d10_hbm,
                    src_v, ones_v, buf_v, hout_sh, sa):
    c = lax.axis_index("c")
    s = lax.axis_index("s")
    w = s * NC + c

    def fill_zero(i, carry):
        buf_v[pl.ds(i * 16, 16)] = jnp.zeros((16,), jnp.float32)
        return carry

    lax.fori_loop(0, DEG_SLICE // 16, fill_zero, 0)

    def fill_one(i, carry):
        ones_v[pl.ds(i * 16, 16)] = jnp.ones((16,), jnp.float32)
        return carry

    lax.fori_loop(0, 128 // 16, fill_one, 0)

    sl = pl.ds(s * DEG_SLICE, DEG_SLICE)
    pltpu.sync_copy(buf_v, hout_sh.at[sl])
    pltpu.sync_copy(e2_hbm.at[0, pl.ds(w * NCHUNK, NCHUNK), :], src_v)
    plsc.subcore_barrier()

    ones_sl = ones_v.at[pl.ds(0, G)]

    def body(j, carry):
        @pl.when(j >= 1)
        def _():
            pltpu.make_async_copy(ones_sl, hout_sh.at[src_v.at[0]], sa).wait()

        pltpu.async_copy(ones_sl, hout_sh.at[src_v.at[j]], sa, add=True)
        return carry

    lax.fori_loop(0, NCHUNK, body, 0)
    pltpu.make_async_copy(ones_sl, hout_sh.at[src_v.at[0]], sa).wait()
    plsc.subcore_barrier()

    @pl.when(c == 0)
    def _():
        pltpu.sync_copy(hout_sh.at[sl], buf_v)
        pltpu.sync_copy(buf_v, d00_hbm.at[sl])

    @pl.when(c == 1)
    def _():
        pltpu.sync_copy(hout_sh.at[sl], buf_v)
        pltpu.sync_copy(buf_v, d10_hbm.at[sl])


@functools.partial(
    pl.kernel,
    out_type=(jax.ShapeDtypeStruct((NC, NPAD, D), jnp.float32),
              jax.ShapeDtypeStruct((NPAD,), jnp.float32),
              jax.ShapeDtypeStruct((NPAD,), jnp.float32)),
    mesh=_mesh,
    scratch_types=[
        pltpu.VMEM((NCHUNK, G), jnp.int32),
        pltpu.VMEM((8, G), jnp.int32),
        tuple(pltpu.VMEM((G, D), jnp.float32) for _ in range(2)),
        pltpu.VMEM((128,), jnp.float32),
        pltpu.VMEM_SHARED((NPAD, D), jnp.float32),
        pltpu.VMEM_SHARED((NPAD,), jnp.float32),
        tuple(pltpu.SemaphoreType.DMA for _ in range(2)),
        tuple(pltpu.SemaphoreType.DMA for _ in range(2)),
        pltpu.SemaphoreType.DMA,
    ],
)
def _aggregate_kernel(feat_hbm, e2_hbm, agg_hbm, d01_hbm, d11_hbm,
                      src_v, dst_v, rows, ones_v, agg_sh, hin_sh,
                      gsem, ssem, hsem):
    c = lax.axis_index("c")
    s = lax.axis_index("s")
    w = s * NC + c

    def fill_row(i, carry):
        def fill_col(k, carry2):
            rows[0][i, pl.ds(k * 16, 16)] = jnp.zeros((16,), jnp.float32)
            return carry2

        lax.fori_loop(0, D // 16, fill_col, 0)
        return carry

    lax.fori_loop(0, G, fill_row, 0)

    def fill_one(i, carry):
        ones_v[pl.ds(i * 16, 16)] = jnp.ones((16,), jnp.float32)
        return carry

    lax.fori_loop(0, 128 // 16, fill_one, 0)

    base_row = s * ROWS_PER_TILE
    for r in range(ROWS_PER_TILE // G):
        pltpu.sync_copy(rows[0], agg_sh.at[pl.ds(base_row + r * G, G), :])
    rem = ROWS_PER_TILE % G
    if rem:
        pltpu.sync_copy(rows[0].at[pl.ds(0, rem), :],
                        agg_sh.at[pl.ds(base_row + (ROWS_PER_TILE // G) * G,
                                        rem), :])
    dsl = pl.ds(s * DEG_SLICE, DEG_SLICE)
    pltpu.sync_copy(rows[0].at[0, pl.ds(0, 128)],
                    hin_sh.at[pl.ds(s * DEG_SLICE, 128)])
    for r in range(1, DEG_SLICE // 128):
        pltpu.sync_copy(rows[0].at[0, pl.ds(0, 128)],
                        hin_sh.at[pl.ds(s * DEG_SLICE + r * 128, 128)])

    pltpu.sync_copy(e2_hbm.at[0, pl.ds(w * NCHUNK, NCHUNK), :], src_v)
    pltpu.sync_copy(e2_hbm.at[1, pl.ds(w * NCHUNK, 8), :], dst_v)
    plsc.subcore_barrier()

    ones_sl = ones_v.at[pl.ds(0, G)]
    pltpu.async_copy(feat_hbm.at[src_v.at[0]], rows[0], gsem[0])

    def body(i, carry):
        for b in range(2):
            t = i * 2 + b
            pltpu.make_async_copy(feat_hbm.at[src_v.at[0]], rows[b],
                                  gsem[b]).wait()

            @pl.when(t >= 1)
            def _():
                pltpu.make_async_copy(rows[1 - b], agg_sh.at[dst_v.at[0]],
                                      ssem[1 - b]).wait()
                pltpu.make_async_copy(ones_sl, hin_sh.at[dst_v.at[0]],
                                      hsem).wait()

            if b == 0:
                @pl.when(jnp.logical_and(t % 8 == 0, t > 0))
                def _():
                    pltpu.sync_copy(
                        e2_hbm.at[1, pl.ds(w * NCHUNK + (t // 8) * 8, 8), :],
                        dst_v)

            @pl.when(t + 1 < NCHUNK)
            def _():
                pltpu.async_copy(feat_hbm.at[src_v.at[t + 1]], rows[1 - b],
                                 gsem[1 - b])

            pltpu.async_copy(rows[b], agg_sh.at[dst_v.at[t % 8]], add=True,
                             sem=ssem[b])
            pltpu.async_copy(ones_sl, hin_sh.at[dst_v.at[t % 8]], hsem,
                             add=True)
        return carry

    lax.fori_loop(0, NCHUNK // 2, body, 0)
    pltpu.make_async_copy(rows[1], agg_sh.at[dst_v.at[0]], ssem[1]).wait()
    pltpu.make_async_copy(ones_sl, hin_sh.at[dst_v.at[0]], hsem).wait()
    plsc.subcore_barrier()

    sl = pl.ds(base_row, ROWS_PER_TILE)
    pltpu.sync_copy(agg_sh.at[sl, :], agg_hbm.at[c, sl, :])

    @pl.when(c == 0)
    def _():
        pltpu.sync_copy(hin_sh.at[dsl], d01_hbm.at[dsl])

    @pl.when(c == 1)
    def _():
        pltpu.sync_copy(hin_sh.at[dsl], d11_hbm.at[dsl])


RB = 2048
NBLK = NPAD // RB


def _feat_body(d00_ref, d10_ref, v_ref, feat_ref):
    d_out = d00_ref[...] + d10_ref[...]
    rs = lax.rsqrt(jnp.maximum(d_out, 1.0))
    feat_ref[...] = v_ref[...] * rs[:, None]


_feat_call = pl.pallas_call(
    _feat_body,
    grid=(NBLK,),
    in_specs=[
        pl.BlockSpec((RB,), lambda i: (i,)),
        pl.BlockSpec((RB,), lambda i: (i,)),
        pl.BlockSpec((RB, D), lambda i: (i, 0)),
    ],
    out_specs=pl.BlockSpec((RB, D), lambda i: (i, 0)),
    out_shape=jax.ShapeDtypeStruct((N, D), jnp.float32),
)


def _out_body(aggp_ref, d01_ref, d11_ref, v_ref, w_ref, b_ref, out_ref):
    agg = aggp_ref[0] + aggp_ref[1]
    d_in = d01_ref[...] + d11_ref[...]
    rs = lax.rsqrt(jnp.maximum(d_in, 1.0))
    rst = agg * rs[:, None]
    rst = jnp.dot(rst, w_ref[...], preferred_element_type=jnp.float32)
    rst = rst + b_ref[...]
    out_ref[...] = jnp.where(rst > 0, rst, SLOPE * rst) + v_ref[...]


_out_call = pl.pallas_call(
    _out_body,
    grid=(NBLK,),
    in_specs=[
        pl.BlockSpec((NC, RB, D), lambda i: (0, i, 0)),
        pl.BlockSpec((RB,), lambda i: (i,)),
        pl.BlockSpec((RB,), lambda i: (i,)),
        pl.BlockSpec((RB, D), lambda i: (i, 0)),
        pl.BlockSpec((D, D), lambda i: (0, 0)),
        pl.BlockSpec((1, D), lambda i: (0, 0)),
    ],
    out_specs=pl.BlockSpec((RB, D), lambda i: (i, 0)),
    out_shape=jax.ShapeDtypeStruct((N, D), jnp.float32),
)


def kernel(V, edge_index, W, b):
    e2 = edge_index.reshape(2, E // G, G)
    d00, d10 = _degrees_kernel(e2)
    feat = _feat_call(d00, d10, V)
    aggp, d01, d11 = _aggregate_kernel(feat, e2)
    return _out_call(aggp, d01, d11, V, W, b.reshape(1, D))


# speedup vs baseline: 1.2434x; 1.0598x over previous
import functools

import jax
import jax.numpy as jnp
from jax import lax
from jax.experimental import pallas as pl
from jax.experimental.pallas import tpu as pltpu
from jax.experimental.pallas import tpu_sc as plsc

N = 10000
E = 320000
D = 128
SLOPE = 0.01

NC, NS = 2, 16
NW = NC * NS
G = 125
EPW = E // NW
NCHUNK = EPW // G
NPAD = 10240
DEG_SLICE = NPAD // NS
ROWS_PER_TILE = NPAD // NS

_mesh = plsc.VectorSubcoreMesh(core_axis_name="c", subcore_axis_name="s")


@functools.partial(
    pl.kernel,
    out_type=tuple(jax.ShapeDtypeStruct((NPAD,), jnp.float32) for _ in range(2)),
    mesh=_mesh,
    scratch_types=[
        pltpu.VMEM((NCHUNK, G), jnp.int32),
        pltpu.VMEM((128,), jnp.float32),
        pltpu.VMEM((DEG_SLICE,), jnp.float32),
        pltpu.VMEM_SHARED((NPAD,), jnp.float32),
        pltpu.SemaphoreType.DMA,
    ],
)
def _degrees_kernel(e2_hbm, d00_hbm, d10_hbm,
                    src_v, ones_v, buf_v, hout_sh, sa):
    c = lax.axis_index("c")
    s = lax.axis_index("s")
    w = s * NC + c

    def fill_zero(i, carry):
        buf_v[pl.ds(i * 16, 16)] = jnp.zeros((16,), jnp.float32)
        return carry

    lax.fori_loop(0, DEG_SLICE // 16, fill_zero, 0)

    def fill_one(i, carry):
        ones_v[pl.ds(i * 16, 16)] = jnp.ones((16,), jnp.float32)
        return carry

    lax.fori_loop(0, 128 // 16, fill_one, 0)

    sl = pl.ds(s * DEG_SLICE, DEG_SLICE)
    pltpu.sync_copy(buf_v, hout_sh.at[sl])
    pltpu.sync_copy(e2_hbm.at[0, pl.ds(w * NCHUNK, NCHUNK), :], src_v)
    plsc.subcore_barrier()

    ones_sl = ones_v.at[pl.ds(0, G)]

    def body(j, carry):
        @pl.when(j >= 4)
        def _():
            pltpu.make_async_copy(ones_sl, hout_sh.at[src_v.at[0]], sa).wait()

        pltpu.async_copy(ones_sl, hout_sh.at[src_v.at[j]], sa, add=True)
        return carry

    lax.fori_loop(0, NCHUNK, body, 0)
    for _ in range(4):
        pltpu.make_async_copy(ones_sl, hout_sh.at[src_v.at[0]], sa).wait()
    plsc.subcore_barrier()

    @pl.when(c == 0)
    def _():
        pltpu.sync_copy(hout_sh.at[sl], buf_v)
        pltpu.sync_copy(buf_v, d00_hbm.at[sl])

    @pl.when(c == 1)
    def _():
        pltpu.sync_copy(hout_sh.at[sl], buf_v)
        pltpu.sync_copy(buf_v, d10_hbm.at[sl])


@functools.partial(
    pl.kernel,
    out_type=(jax.ShapeDtypeStruct((NC, NPAD, D), jnp.float32),
              jax.ShapeDtypeStruct((NPAD,), jnp.float32),
              jax.ShapeDtypeStruct((NPAD,), jnp.float32)),
    mesh=_mesh,
    scratch_types=[
        pltpu.VMEM((NCHUNK, G), jnp.int32),
        pltpu.VMEM((2, 8, G), jnp.int32),
        tuple(pltpu.VMEM((G, D), jnp.float32) for _ in range(2)),
        pltpu.VMEM((128,), jnp.float32),
        pltpu.VMEM_SHARED((NPAD, D), jnp.float32),
        pltpu.VMEM_SHARED((NPAD,), jnp.float32),
        tuple(pltpu.SemaphoreType.DMA for _ in range(2)),
        tuple(pltpu.SemaphoreType.DMA for _ in range(2)),
        pltpu.SemaphoreType.DMA,
        pltpu.SemaphoreType.DMA,
    ],
)
def _aggregate_kernel(feat_hbm, e2_hbm, agg_hbm, d01_hbm, d11_hbm,
                      src_v, dst_v, rows, ones_v, agg_sh, hin_sh,
                      gsem, ssem, hsem, dsem):
    c = lax.axis_index("c")
    s = lax.axis_index("s")
    w = s * NC + c

    def fill_row(i, carry):
        def fill_col(k, carry2):
            rows[0][i, pl.ds(k * 16, 16)] = jnp.zeros((16,), jnp.float32)
            return carry2

        lax.fori_loop(0, D // 16, fill_col, 0)
        return carry

    lax.fori_loop(0, G, fill_row, 0)

    def fill_one(i, carry):
        ones_v[pl.ds(i * 16, 16)] = jnp.ones((16,), jnp.float32)
        return carry

    lax.fori_loop(0, 128 // 16, fill_one, 0)

    base_row = s * ROWS_PER_TILE
    for r in range(ROWS_PER_TILE // G):
        pltpu.sync_copy(rows[0], agg_sh.at[pl.ds(base_row + r * G, G), :])
    rem = ROWS_PER_TILE % G
    if rem:
        pltpu.sync_copy(rows[0].at[pl.ds(0, rem), :],
                        agg_sh.at[pl.ds(base_row + (ROWS_PER_TILE // G) * G,
                                        rem), :])
    dsl = pl.ds(s * DEG_SLICE, DEG_SLICE)
    pltpu.sync_copy(rows[0].at[0, pl.ds(0, 128)],
                    hin_sh.at[pl.ds(s * DEG_SLICE, 128)])
    for r in range(1, DEG_SLICE // 128):
        pltpu.sync_copy(rows[0].at[0, pl.ds(0, 128)],
                        hin_sh.at[pl.ds(s * DEG_SLICE + r * 128, 128)])

    pltpu.sync_copy(e2_hbm.at[0, pl.ds(w * NCHUNK, NCHUNK), :], src_v)
    pltpu.sync_copy(e2_hbm.at[1, pl.ds(w * NCHUNK, 8), :], dst_v.at[0])
    pltpu.async_copy(e2_hbm.at[1, pl.ds(w * NCHUNK + 8, 8), :], dst_v.at[1],
                     dsem)
    plsc.subcore_barrier()

    ones_sl = ones_v.at[pl.ds(0, G)]
    pltpu.async_copy(feat_hbm.at[src_v.at[0]], rows[0], gsem[0])

    def body(i, carry):
        for b in range(2):
            t = i * 2 + b
            q = t // 8
            pltpu.make_async_copy(feat_hbm.at[src_v.at[0]], rows[b],
                                  gsem[b]).wait()

            @pl.when(t >= 1)
            def _():
                pltpu.make_async_copy(rows[1 - b], agg_sh.at[dst_v.at[0, 0]],
                                      ssem[1 - b]).wait()
                pltpu.make_async_copy(ones_sl, hin_sh.at[dst_v.at[0, 0]],
                                      hsem).wait()

            if b == 0:
                @pl.when(jnp.logical_and(t % 8 == 0, t > 0))
                def _():
                    pltpu.make_async_copy(
                        e2_hbm.at[1, pl.ds(w * NCHUNK, 8), :],
                        dst_v.at[0], dsem).wait()

                    @pl.when(q + 1 < NCHUNK // 8)
                    def _():
                        pltpu.async_copy(
                            e2_hbm.at[1,
                                      pl.ds(w * NCHUNK + (q + 1) * 8, 8), :],
                            dst_v.at[(q + 1) % 2], dsem)

            @pl.when(t + 1 < NCHUNK)
            def _():
                pltpu.async_copy(feat_hbm.at[src_v.at[t + 1]], rows[1 - b],
                                 gsem[1 - b])

            idx = dst_v.at[q % 2, t % 8]
            pltpu.async_copy(rows[b], agg_sh.at[idx], add=True, sem=ssem[b])
            pltpu.async_copy(ones_sl, hin_sh.at[idx], hsem, add=True)
        return carry

    lax.fori_loop(0, NCHUNK // 2, body, 0)
    pltpu.make_async_copy(rows[1], agg_sh.at[dst_v.at[0, 0]], ssem[1]).wait()
    pltpu.make_async_copy(ones_sl, hin_sh.at[dst_v.at[0, 0]], hsem).wait()
    plsc.subcore_barrier()

    sl = pl.ds(base_row, ROWS_PER_TILE)
    pltpu.sync_copy(agg_sh.at[sl, :], agg_hbm.at[c, sl, :])

    @pl.when(c == 0)
    def _():
        pltpu.sync_copy(hin_sh.at[dsl], d01_hbm.at[dsl])

    @pl.when(c == 1)
    def _():
        pltpu.sync_copy(hin_sh.at[dsl], d11_hbm.at[dsl])


RB = 2048
NBLK = NPAD // RB


def _feat_body(d00_ref, d10_ref, v_ref, feat_ref):
    d_out = d00_ref[...] + d10_ref[...]
    rs = lax.rsqrt(jnp.maximum(d_out, 1.0))
    feat_ref[...] = v_ref[...] * rs[:, None]


_feat_call = pl.pallas_call(
    _feat_body,
    grid=(NBLK,),
    in_specs=[
        pl.BlockSpec((RB,), lambda i: (i,)),
        pl.BlockSpec((RB,), lambda i: (i,)),
        pl.BlockSpec((RB, D), lambda i: (i, 0)),
    ],
    out_specs=pl.BlockSpec((RB, D), lambda i: (i, 0)),
    out_shape=jax.ShapeDtypeStruct((N, D), jnp.float32),
)


def _out_body(aggp_ref, d01_ref, d11_ref, v_ref, w_ref, b_ref, out_ref):
    agg = aggp_ref[0] + aggp_ref[1]
    d_in = d01_ref[...] + d11_ref[...]
    rs = lax.rsqrt(jnp.maximum(d_in, 1.0))
    rst = agg * rs[:, None]
    rst = jnp.dot(rst, w_ref[...], preferred_element_type=jnp.float32)
    rst = rst + b_ref[...]
    out_ref[...] = jnp.where(rst > 0, rst, SLOPE * rst) + v_ref[...]


_out_call = pl.pallas_call(
    _out_body,
    grid=(NBLK,),
    in_specs=[
        pl.BlockSpec((NC, RB, D), lambda i: (0, i, 0)),
        pl.BlockSpec((RB,), lambda i: (i,)),
        pl.BlockSpec((RB,), lambda i: (i,)),
        pl.BlockSpec((RB, D), lambda i: (i, 0)),
        pl.BlockSpec((D, D), lambda i: (0, 0)),
        pl.BlockSpec((1, D), lambda i: (0, 0)),
    ],
    out_specs=pl.BlockSpec((RB, D), lambda i: (i, 0)),
    out_shape=jax.ShapeDtypeStruct((N, D), jnp.float32),
)


def kernel(V, edge_index, W, b):
    e2 = edge_index.reshape(2, E // G, G)
    d00, d10 = _degrees_kernel(e2)
    feat = _feat_call(d00, d10, V)
    aggp, d01, d11 = _aggregate_kernel(feat, e2)
    return _out_call(aggp, d01, d11, V, W, b.reshape(1, D))
